# baseline (speedup 1.0000x reference)
import jax
import jax.numpy as jnp
from jax.experimental import pallas as pl
from jax.experimental.pallas import tpu as pltpu

B = 32
SEQ = 3000
ENC = 256
NC = 5
RW = 256
NE = 4

CHUNK = 512
BOUNDS = (0, 512, 1024, 1536, 2048, 2560, 3000)
NCH = len(BOUNDS) - 1
NS = 8
NBUF = 16


def _moe_body(eeg, eog, w0, _w0d, w1, _w1d, w2a, _w2ad, w2b, _w2bd,
              w3a, _w3ad, w3b, _w3bd, wr1a, wr1b,
              wh0t, wh1t, wh2t, wh3t, b1, wr2t, b2,
              out_logits, out_rw, out_eo, out_il,
              buf, sem, a0, a1, a2a, a2b, a3a, a3b, ar):
    streams = ((w0, 0, eeg, a0), (w1, 0, eog, a1),
               (w2a, 0, eeg, a2a), (w2b, 0, eog, a2b),
               (w3a, 0, eeg, a3a), (w3b, 0, eog, a3b),
               (wr1a, 0, eeg, ar), (wr1b, SEQ, eog, ar))
    chunks = [(s, c) for c in range(NCH) for s in range(NS)]

    def copy_desc(idx, slot):
        s, c = chunks[idx]
        wref, base, _, _ = streams[s]
        r0, r1 = BOUNDS[c], BOUNDS[c + 1]
        rows = r1 - r0
        return pltpu.make_async_copy(
            wref.at[pl.ds(base + r0, rows), :],
            buf.at[slot, pl.ds(0, rows), :],
            sem.at[slot])

    for i in range(NBUF):
        copy_desc(i, i % NBUF).start()

    def dot(a, b):
        return jnp.dot(a, b, preferred_element_type=jnp.float32)

    zacc = jnp.zeros((B, ENC), jnp.float32)
    a0[...] = zacc
    a1[...] = zacc
    a2a[...] = zacc
    a2b[...] = zacc
    a3a[...] = zacc
    a3b[...] = zacc
    ar[...] = zacc

    for idx, (s, c) in enumerate(chunks):
        slot = idx % NBUF
        copy_desc(idx, slot).wait()
        _, _, act, acc = streams[s]
        r0, r1 = BOUNDS[c], BOUNDS[c + 1]
        rows = r1 - r0
        acc[...] += dot(act[:, r0:r1], buf[slot, 0:rows, :])
        nxt = idx + NBUF
        if nxt < len(chunks):
            copy_desc(nxt, nxt % NBUF).start()

    def dott(a, bt):
        return jax.lax.dot_general(
            a, bt, (((1,), (1,)), ((), ())),
            preferred_element_type=jnp.float32)

    relu = lambda t: jnp.maximum(t, 0.0)
    h0 = relu(a0[...])
    h1 = relu(a1[...])
    h2a = relu(a2a[...])
    h2b = relu(a2b[...])
    h3a = relu(a3a[...])
    h3b = relu(a3b[...])
    eo0 = dott(h0, wh0t[...])
    eo1 = dott(h1, wh1t[...])
    eo2 = dott(h2a, wh2t[:, 0:ENC]) + dott(h2b, wh2t[:, ENC:2 * ENC])
    eo3 = dott(h3a, wh3t[:, 0:ENC]) + dott(h3b, wh3t[:, ENC:2 * ENC])
    hr = relu(ar[...] + b1[...])
    rl = dott(hr, wr2t[...]) + b2[...]
    m = jnp.max(rl, axis=-1, keepdims=True)
    ex = jnp.exp(rl - m)
    rw = ex / jnp.sum(ex, axis=-1, keepdims=True)
    out_rw[...] = rw.T
    logits = (rw[:, 0:1] * eo0 + rw[:, 1:2] * eo1
              + rw[:, 2:3] * eo2 + rw[:, 3:4] * eo3)
    out_logits[...] = logits.T
    out_eo[...] = jnp.stack([eo0.T, eo1.T, eo2.T, eo3.T], axis=1)
    avg = 0.25 * (eo0 + eo1 + eo2 + eo3)
    inv = 1.0 / (B * NC)
    out_il[0:1, 0:1] = (jnp.sum((eo0 - avg) ** 2, keepdims=True) * inv)
    out_il[0:1, 1:2] = (jnp.sum((eo1 - avg) ** 2, keepdims=True) * inv)
    out_il[0:1, 2:3] = (jnp.sum((eo2 - avg) ** 2, keepdims=True) * inv)
    out_il[0:1, 3:4] = (jnp.sum((eo3 - avg) ** 2, keepdims=True) * inv)


def kernel(eeg, eog, We_eeg0, We_eog0, Wh0, We_eeg1, We_eog1, Wh1,
           We_eeg2, We_eog2, Wh2, We_eeg3, We_eog3, Wh3,
           Wr1, br1, Wr2, br2):
    b1 = br1.reshape(1, RW)
    b2 = br2.reshape(1, NE)

    hbm = pl.BlockSpec(memory_space=pltpu.MemorySpace.HBM)
    vmem = pl.BlockSpec(memory_space=pltpu.MemorySpace.VMEM)

    out_shape = (
        jax.ShapeDtypeStruct((NC, B), jnp.float32),
        jax.ShapeDtypeStruct((NE, B), jnp.float32),
        jax.ShapeDtypeStruct((NC, NE, B), jnp.float32),
        jax.ShapeDtypeStruct((1, NE), jnp.float32),
    )

    logitsT, rwT, eoP, il = pl.pallas_call(
        _moe_body,
        in_specs=[vmem, vmem,
                  hbm, hbm, hbm, hbm, hbm, hbm, hbm, hbm,
                  hbm, hbm, hbm, hbm, hbm, hbm,
                  vmem, vmem, vmem, vmem, vmem, vmem, vmem],
        out_specs=(vmem, vmem, vmem, vmem),
        out_shape=out_shape,
        scratch_shapes=[pltpu.VMEM((NBUF, CHUNK, ENC), jnp.float32),
                        pltpu.SemaphoreType.DMA((NBUF,))]
        + [pltpu.VMEM((B, ENC), jnp.float32)] * 7,
        compiler_params=pltpu.CompilerParams(
            vmem_limit_bytes=62 * 1024 * 1024),
    )(eeg, eog, We_eeg0, We_eeg0, We_eog1, We_eog1,
      We_eeg2, We_eeg2, We_eog2, We_eog2, We_eeg3, We_eeg3,
      We_eog3, We_eog3, Wr1, Wr1,
      Wh0.T, Wh1.T, Wh2.T, Wh3.T, b1, Wr2.T, b2)
    return (logitsT.T, rwT.T, jnp.transpose(eoP, (1, 2, 0)),
            il.reshape(NE))


# speedup vs baseline: 1.0315x; 1.0315x over previous
import jax
import jax.numpy as jnp
from jax.experimental import pallas as pl
from jax.experimental.pallas import tpu as pltpu

B = 32
SEQ = 3000
ENC = 256
NC = 5
RW = 256
NE = 4
NS = 8


def _moe_body(eeg, eog, w0, _w0d, w1, _w1d, w2a, _w2ad, w2b, _w2bd,
              w3a, _w3ad, w3b, _w3bd, wr1a, wr1b,
              wh0t, wh1t, wh2t, wh3t, b1, wr2t, b2,
              out_logits, out_rw, out_eo, out_il,
              buf, sem, a0, a1, a2a, a2b, a3a, a3b, ar):
    streams = ((w0, 0, eeg, a0), (w1, 0, eog, a1),
               (w2a, 0, eeg, a2a), (w2b, 0, eog, a2b),
               (w3a, 0, eeg, a3a), (w3b, 0, eog, a3b),
               (wr1a, 0, eeg, ar), (wr1b, SEQ, eog, ar))

    def copy_desc(s):
        wref, base, _, _ = streams[s]
        return pltpu.make_async_copy(
            wref.at[pl.ds(base, SEQ), :],
            buf.at[s],
            sem.at[s])

    for s in range(NS):
        copy_desc(s).start()

    def dot(a, b):
        return jnp.dot(a, b, preferred_element_type=jnp.float32)

    seen = set()
    for s in range(NS):
        copy_desc(s).wait()
        _, _, act, acc = streams[s]
        val = dot(act[...], buf[s])
        if id(acc) in seen:
            acc[...] += val
        else:
            acc[...] = val
            seen.add(id(acc))

    def dott(a, bt):
        return jax.lax.dot_general(
            a, bt, (((1,), (1,)), ((), ())),
            preferred_element_type=jnp.float32)

    relu = lambda t: jnp.maximum(t, 0.0)
    h0 = relu(a0[...])
    h1 = relu(a1[...])
    h2a = relu(a2a[...])
    h2b = relu(a2b[...])
    h3a = relu(a3a[...])
    h3b = relu(a3b[...])
    eo0 = dott(h0, wh0t[...])
    eo1 = dott(h1, wh1t[...])
    eo2 = dott(h2a, wh2t[:, 0:ENC]) + dott(h2b, wh2t[:, ENC:2 * ENC])
    eo3 = dott(h3a, wh3t[:, 0:ENC]) + dott(h3b, wh3t[:, ENC:2 * ENC])
    hr = relu(ar[...] + b1[...])
    rl = dott(hr, wr2t[...]) + b2[...]
    m = jnp.max(rl, axis=-1, keepdims=True)
    ex = jnp.exp(rl - m)
    rw = ex / jnp.sum(ex, axis=-1, keepdims=True)
    out_rw[...] = rw.T
    logits = (rw[:, 0:1] * eo0 + rw[:, 1:2] * eo1
              + rw[:, 2:3] * eo2 + rw[:, 3:4] * eo3)
    out_logits[...] = logits.T
    out_eo[...] = jnp.stack([eo0.T, eo1.T, eo2.T, eo3.T], axis=1)
    avg = 0.25 * (eo0 + eo1 + eo2 + eo3)
    inv = 1.0 / (B * NC)
    out_il[0:1, 0:1] = (jnp.sum((eo0 - avg) ** 2, keepdims=True) * inv)
    out_il[0:1, 1:2] = (jnp.sum((eo1 - avg) ** 2, keepdims=True) * inv)
    out_il[0:1, 2:3] = (jnp.sum((eo2 - avg) ** 2, keepdims=True) * inv)
    out_il[0:1, 3:4] = (jnp.sum((eo3 - avg) ** 2, keepdims=True) * inv)


def kernel(eeg, eog, We_eeg0, We_eog0, Wh0, We_eeg1, We_eog1, Wh1,
           We_eeg2, We_eog2, Wh2, We_eeg3, We_eog3, Wh3,
           Wr1, br1, Wr2, br2):
    b1 = br1.reshape(1, RW)
    b2 = br2.reshape(1, NE)

    hbm = pl.BlockSpec(memory_space=pltpu.MemorySpace.HBM)
    vmem = pl.BlockSpec(memory_space=pltpu.MemorySpace.VMEM)

    out_shape = (
        jax.ShapeDtypeStruct((NC, B), jnp.float32),
        jax.ShapeDtypeStruct((NE, B), jnp.float32),
        jax.ShapeDtypeStruct((NC, NE, B), jnp.float32),
        jax.ShapeDtypeStruct((1, NE), jnp.float32),
    )

    logitsT, rwT, eoP, il = pl.pallas_call(
        _moe_body,
        in_specs=[vmem, vmem,
                  hbm, hbm, hbm, hbm, hbm, hbm, hbm, hbm,
                  hbm, hbm, hbm, hbm, hbm, hbm,
                  vmem, vmem, vmem, vmem, vmem, vmem, vmem],
        out_specs=(vmem, vmem, vmem, vmem),
        out_shape=out_shape,
        scratch_shapes=[pltpu.VMEM((NS, SEQ, ENC), jnp.float32),
                        pltpu.SemaphoreType.DMA((NS,))]
        + [pltpu.VMEM((B, ENC), jnp.float32)] * 7,
        compiler_params=pltpu.CompilerParams(
            vmem_limit_bytes=62 * 1024 * 1024),
    )(eeg, eog, We_eeg0, We_eeg0, We_eog1, We_eog1,
      We_eeg2, We_eeg2, We_eog2, We_eog2, We_eeg3, We_eeg3,
      We_eog3, We_eog3, Wr1, Wr1,
      Wh0.T, Wh1.T, Wh2.T, Wh3.T, b1, Wr2.T, b2)
    return (logitsT.T, rwT.T, jnp.transpose(eoP, (1, 2, 0)),
            il.reshape(NE))


# speedup vs baseline: 1.0461x; 1.0141x over previous
import jax
import jax.numpy as jnp
from jax.experimental import pallas as pl
from jax.experimental.pallas import tpu as pltpu

B = 32
SEQ = 3000
ENC = 256
NC = 5
RW = 256
NE = 4
NS = 8


def _moe_body(eeg, eog, w0, _w0d, w1, _w1d, w2a, _w2ad, w2b, _w2bd,
              w3a, _w3ad, w3b, _w3bd, wr1a, wr1b,
              wh0t, wh1t, wh2t, wh3t, b1, wr2t, b2,
              out_logits, out_rw, out_eo, out_il,
              buf, xa, xb, sem, a0, a1, a2a, a2b, a3a, a3b, ar):
    streams = ((w0, 0, xa, a0), (w1, 0, xb, a1),
               (w2a, 0, xa, a2a), (w2b, 0, xb, a2b),
               (w3a, 0, xa, a3a), (w3b, 0, xb, a3b),
               (wr1a, 0, xa, ar), (wr1b, SEQ, xb, ar))

    def copy_desc(s):
        wref, base, _, _ = streams[s]
        return pltpu.make_async_copy(
            wref.at[pl.ds(base, SEQ), :],
            buf.at[s],
            sem.at[s])

    cpa = pltpu.make_async_copy(eeg, xa, sem.at[NS])
    cpb = pltpu.make_async_copy(eog, xb, sem.at[NS + 1])
    cpa.start()
    cpb.start()
    for s in range(NS):
        copy_desc(s).start()
    cpa.wait()
    cpb.wait()

    def dot(a, b):
        return jnp.dot(a, b, preferred_element_type=jnp.float32)

    def dott(a, bt):
        return jax.lax.dot_general(
            a, bt, (((1,), (1,)), ((), ())),
            preferred_element_type=jnp.float32)

    relu = lambda t: jnp.maximum(t, 0.0)

    def consume(s, acc, add):
        copy_desc(s).wait()
        _, _, act, _ = streams[s]
        val = dot(act[...], buf[s])
        if add:
            acc[...] += val
        else:
            acc[...] = val

    consume(0, a0, False)
    consume(1, a1, False)
    eo0 = dott(relu(a0[...]), wh0t[...])
    eo1 = dott(relu(a1[...]), wh1t[...])

    consume(2, a2a, False)
    consume(3, a2b, False)
    eo2 = (dott(relu(a2a[...]), wh2t[:, 0:ENC])
           + dott(relu(a2b[...]), wh2t[:, ENC:2 * ENC]))

    consume(4, a3a, False)
    consume(5, a3b, False)
    eo3 = (dott(relu(a3a[...]), wh3t[:, 0:ENC])
           + dott(relu(a3b[...]), wh3t[:, ENC:2 * ENC]))

    out_eo[...] = jnp.stack([eo0.T, eo1.T, eo2.T, eo3.T], axis=1)
    avg = 0.25 * (eo0 + eo1 + eo2 + eo3)
    inv = 1.0 / (B * NC)
    out_il[0:1, 0:1] = (jnp.sum((eo0 - avg) ** 2, keepdims=True) * inv)
    out_il[0:1, 1:2] = (jnp.sum((eo1 - avg) ** 2, keepdims=True) * inv)
    out_il[0:1, 2:3] = (jnp.sum((eo2 - avg) ** 2, keepdims=True) * inv)
    out_il[0:1, 3:4] = (jnp.sum((eo3 - avg) ** 2, keepdims=True) * inv)

    consume(6, ar, False)
    consume(7, ar, True)
    hr = relu(ar[...] + b1[...])
    rl = dott(hr, wr2t[...]) + b2[...]
    m = jnp.max(rl, axis=-1, keepdims=True)
    ex = jnp.exp(rl - m)
    rw = ex / jnp.sum(ex, axis=-1, keepdims=True)
    out_rw[...] = rw.T
    logits = (rw[:, 0:1] * eo0 + rw[:, 1:2] * eo1
              + rw[:, 2:3] * eo2 + rw[:, 3:4] * eo3)
    out_logits[...] = logits.T


def kernel(eeg, eog, We_eeg0, We_eog0, Wh0, We_eeg1, We_eog1, Wh1,
           We_eeg2, We_eog2, Wh2, We_eeg3, We_eog3, Wh3,
           Wr1, br1, Wr2, br2):
    b1 = br1.reshape(1, RW)
    b2 = br2.reshape(1, NE)

    hbm = pl.BlockSpec(memory_space=pltpu.MemorySpace.HBM)
    vmem = pl.BlockSpec(memory_space=pltpu.MemorySpace.VMEM)

    out_shape = (
        jax.ShapeDtypeStruct((NC, B), jnp.float32),
        jax.ShapeDtypeStruct((NE, B), jnp.float32),
        jax.ShapeDtypeStruct((NC, NE, B), jnp.float32),
        jax.ShapeDtypeStruct((1, NE), jnp.float32),
    )

    logitsT, rwT, eoP, il = pl.pallas_call(
        _moe_body,
        in_specs=[hbm, hbm,
                  hbm, hbm, hbm, hbm, hbm, hbm, hbm, hbm,
                  hbm, hbm, hbm, hbm, hbm, hbm,
                  vmem, vmem, vmem, vmem, vmem, vmem, vmem],
        out_specs=(vmem, vmem, vmem, vmem),
        out_shape=out_shape,
        scratch_shapes=[pltpu.VMEM((NS, SEQ, ENC), jnp.float32),
                        pltpu.VMEM((B, SEQ), jnp.float32),
                        pltpu.VMEM((B, SEQ), jnp.float32),
                        pltpu.SemaphoreType.DMA((NS + 2,))]
        + [pltpu.VMEM((B, ENC), jnp.float32)] * 7,
        compiler_params=pltpu.CompilerParams(
            vmem_limit_bytes=62 * 1024 * 1024),
    )(eeg, eog, We_eeg0, We_eeg0, We_eog1, We_eog1,
      We_eeg2, We_eeg2, We_eog2, We_eog2, We_eeg3, We_eeg3,
      We_eog3, We_eog3, Wr1, Wr1,
      Wh0.T, Wh1.T, Wh2.T, Wh3.T, b1, Wr2.T, b2)
    return (logitsT.T, rwT.T, jnp.transpose(eoP, (1, 2, 0)),
            il.reshape(NE))
